# Initial kernel scaffold; baseline (speedup 1.0000x reference)
import jax
import jax.numpy as jnp
from jax.experimental import pallas as pl


def kernel(n_feat, edge_index, W_in, W_out):
    raise NotImplementedError("write your pallas kernel here")


# speedup vs baseline: 11.1403x; 11.1403x over previous
import functools

import jax
import jax.numpy as jnp
from jax import lax
from jax.experimental import pallas as pl
from jax.experimental.pallas import tpu as pltpu
from jax.experimental.pallas import tpu_sc as plsc

N = 10000
E = 320000
D = 128
INIT_W = 0.9
W1 = 1.0 - INIT_W

NC = 2
NS = 16
NW = NC * NS
CH = 128
EPW = E // NW
NCH = 80
EPWP = NCH * CH
NP = 10240
RPS = NP // NS

_mesh = plsc.VectorSubcoreMesh(
    core_axis_name="c", subcore_axis_name="s", num_cores=NC, num_subcores=NS
)


@functools.partial(
    pl.kernel,
    out_type=[
        jax.ShapeDtypeStruct((NC, NP), jnp.float32),
        jax.ShapeDtypeStruct((NC, NP), jnp.float32),
    ],
    mesh=_mesh,
    scratch_types=[
        pltpu.VMEM((NCH, CH), jnp.int32),
        pltpu.VMEM((NCH, CH), jnp.int32),
        pltpu.VMEM((CH,), jnp.float32),
        pltpu.VMEM_SHARED((NP,), jnp.float32),
        pltpu.VMEM_SHARED((NP,), jnp.float32),
    ],
)
def _deg(srcs_hbm, dsts_hbm, zvec_hbm, dego_hbm, degi_hbm,
         idx_s, idx_d, ones_v, acc_o, acc_i):
    cid = lax.axis_index("c")
    sid = lax.axis_index("s")
    wid = cid * NS + sid
    rows = pl.ds(sid * RPS, RPS)
    pltpu.sync_copy(zvec_hbm.at[rows], acc_o.at[rows])
    pltpu.sync_copy(zvec_hbm.at[rows], acc_i.at[rows])
    for j in range(CH // 16):
        ones_v[pl.ds(16 * j, 16)] = jnp.ones((16,), jnp.float32)
    pltpu.sync_copy(srcs_hbm.at[wid], idx_s)
    pltpu.sync_copy(dsts_hbm.at[wid], idx_d)
    plsc.subcore_barrier()

    def body(c, carry):
        pltpu.sync_copy(ones_v, acc_o.at[idx_s.at[c]], add=True)
        pltpu.sync_copy(ones_v, acc_i.at[idx_d.at[c]], add=True)
        return carry

    lax.fori_loop(0, NCH, body, 0)
    plsc.subcore_barrier()
    pltpu.sync_copy(acc_o.at[rows], dego_hbm.at[cid, rows])
    pltpu.sync_copy(acc_i.at[rows], degi_hbm.at[cid, rows])


HALF = NCH // 2


@functools.partial(
    pl.kernel,
    out_type=jax.ShapeDtypeStruct((NC, NP, D), jnp.float32),
    mesh=_mesh,
    scratch_types=[
        pltpu.VMEM((HALF, CH), jnp.int32),
        pltpu.VMEM((HALF, CH), jnp.int32),
        pltpu.VMEM((2, CH, D), jnp.float32),
        pltpu.VMEM_SHARED((NP, D), jnp.float32),
        pltpu.SemaphoreType.DMA,
        pltpu.SemaphoreType.DMA,
    ],
)
def _hop(x_hbm, srcs_hbm, dsts_hbm, zrow_hbm, out_hbm,
         idx_s, idx_d, buf, acc, sem0, sem1):
    cid = lax.axis_index("c")
    sid = lax.axis_index("s")
    wid = cid * NS + sid
    rows = pl.ds(sid * RPS, RPS)
    pltpu.sync_copy(zrow_hbm, acc.at[rows])
    plsc.subcore_barrier()

    for h in range(2):
        pltpu.sync_copy(srcs_hbm.at[wid, pl.ds(h * HALF, HALF)], idx_s)
        pltpu.sync_copy(dsts_hbm.at[wid, pl.ds(h * HALF, HALF)], idx_d)
        pltpu.async_copy(x_hbm.at[idx_s.at[0]], buf.at[0], sem0)

        def body(i, carry):
            c0 = 2 * i
            pltpu.async_copy(x_hbm.at[idx_s.at[c0 + 1]], buf.at[1], sem1)
            pltpu.make_async_copy(
                x_hbm.at[idx_s.at[c0]], buf.at[0], sem0).wait()
            pltpu.sync_copy(buf.at[0], acc.at[idx_d.at[c0]], add=True)

            @pl.when(i + 1 < HALF // 2)
            def _():
                pltpu.async_copy(x_hbm.at[idx_s.at[c0 + 2]], buf.at[0], sem0)

            pltpu.make_async_copy(
                x_hbm.at[idx_s.at[c0 + 1]], buf.at[1], sem1).wait()
            pltpu.sync_copy(buf.at[1], acc.at[idx_d.at[c0 + 1]], add=True)
            return carry

        lax.fori_loop(0, HALF // 2, body, 0)

    plsc.subcore_barrier()
    pltpu.sync_copy(acc.at[rows], out_hbm.at[cid, rows])


BR = 1280
G = NP // BR


def _row_spec(shape):
    if len(shape) == 3:
        return pl.BlockSpec((shape[0], BR, shape[2]), lambda i: (0, i, 0))
    if shape == (NP, 1):
        return pl.BlockSpec((BR, 1), lambda i: (i, 0))
    if shape == (NC, NP):
        return pl.BlockSpec((NC, BR), lambda i: (0, i))
    return pl.BlockSpec((BR, shape[1]), lambda i: (i, 0))


def _full_spec(shape):
    return pl.BlockSpec(shape, lambda i: tuple(0 for _ in shape))


def _norm_body(dego_ref, degi_ref, nf_ref, nout_ref, nin_ref, x0_ref):
    do = dego_ref[0, :] + dego_ref[1, :]
    di = degi_ref[0, :] + degi_ref[1, :]
    no = lax.rsqrt(jnp.maximum(do, 1.0))[:, None]
    ni = lax.rsqrt(jnp.maximum(di, 1.0))[:, None]
    nout_ref[...] = no
    nin_ref[...] = ni
    x0_ref[...] = nf_ref[...] * no


def _norm_call(dego, degi, nf):
    return pl.pallas_call(
        _norm_body,
        grid=(G,),
        in_specs=[_row_spec((NC, NP)), _row_spec((NC, NP)),
                  _row_spec((NP, D))],
        out_specs=[_row_spec((NP, 1)), _row_spec((NP, 1)),
                   _row_spec((NP, D))],
        out_shape=[
            jax.ShapeDtypeStruct((NP, 1), jnp.float32),
            jax.ShapeDtypeStruct((NP, 1), jnp.float32),
            jax.ShapeDtypeStruct((NP, D), jnp.float32),
        ],
    )(dego, degi, nf)


def _comb_body(p_ref, nin_ref, nout_ref, nf_ref, l_ref, xt_ref):
    agg = p_ref[0] + p_ref[1]
    layer = W1 * (agg * nin_ref[...]) + INIT_W * nf_ref[...]
    l_ref[...] = layer
    xt_ref[...] = layer * nout_ref[...]


def _comb_call(p, nin, nout, nf):
    return pl.pallas_call(
        _comb_body,
        grid=(G,),
        in_specs=[_row_spec((NC, NP, D)), _row_spec((NP, 1)),
                  _row_spec((NP, 1)), _row_spec((NP, D))],
        out_specs=[_row_spec((NP, D)), _row_spec((NP, D))],
        out_shape=[
            jax.ShapeDtypeStruct((NP, D), jnp.float32),
            jax.ShapeDtypeStruct((NP, D), jnp.float32),
        ],
    )(p, nin, nout, nf)


def _final_body(p_ref, nin_ref, nf_ref, l1_ref, l2_ref, win_ref, wout_ref,
                out_ref):
    f32 = jnp.float32
    bf16 = jnp.bfloat16
    agg = p_ref[0] + p_ref[1]
    l0 = nf_ref[...]
    l1 = l1_ref[...]
    l2 = l2_ref[...]
    l3 = W1 * (agg * nin_ref[...]) + INIT_W * l0

    wq_b = win_ref[0:D, :].astype(bf16)
    wk_b = win_ref[D:2 * D, :].astype(bf16)
    wv_b = win_ref[2 * D:3 * D, :].astype(bf16)
    wo_b = wout_ref[...].astype(bf16)
    scale = 1.0 / (128.0 ** 0.5)
    q0 = jnp.sum(wq_b.astype(f32), axis=1, keepdims=True)
    qsc_b = (q0 * scale).astype(bf16)
    u_row = jnp.sum(wo_b.astype(f32), axis=0, keepdims=True)

    dn_t = (((1,), (1,)), ((), ()))
    dn_v = (((1,), (0,)), ((), ()))

    def kv(l):
        l_b = l.astype(bf16)
        k_t = lax.dot_general(l_b, wk_b, dn_t, preferred_element_type=f32)
        v_t = lax.dot_general(l_b, wv_b, dn_t, preferred_element_type=f32)
        lg = lax.dot_general(k_t.astype(bf16), qsc_b, dn_v,
                             preferred_element_type=f32)
        return lg, v_t.astype(bf16).astype(f32)

    lg0, v0 = kv(l0)
    lg1, v1 = kv(l1)
    lg2, v2 = kv(l2)
    lg3, v3 = kv(l3)
    m = jnp.maximum(jnp.maximum(lg0, lg1), jnp.maximum(lg2, lg3))
    e0 = jnp.exp(lg0 - m)
    e1 = jnp.exp(lg1 - m)
    e2 = jnp.exp(lg2 - m)
    e3 = jnp.exp(lg3 - m)
    den = e0 + e1 + e2 + e3

    def wtr(e):
        return (e / den).astype(bf16).astype(f32)

    ctx = wtr(e0) * v0 + wtr(e1) * v1 + wtr(e2) * v2 + wtr(e3) * v3
    c = jnp.sum(ctx.astype(bf16).astype(f32) * u_row, axis=1, keepdims=True)
    s = c / jnp.maximum(jnp.sqrt(4.0 * (c * c)), 1e-12)
    out_ref[...] = s * (((l0 + l1) + l2) + l3)


def _final_call(p3, nin, nf, l1, l2, w_in, w_out):
    return pl.pallas_call(
        _final_body,
        grid=(G,),
        in_specs=[_row_spec((NC, NP, D)), _row_spec((NP, 1)),
                  _row_spec((NP, D)), _row_spec((NP, D)), _row_spec((NP, D)),
                  _full_spec((3 * D, D)), _full_spec((D, D))],
        out_specs=_row_spec((NP, D)),
        out_shape=jax.ShapeDtypeStruct((NP, D), jnp.float32),
    )(p3, nin, nf, l1, l2, w_in, w_out)


def kernel(n_feat, edge_index, W_in, W_out):
    src = edge_index[0]
    dst = edge_index[1]
    pad = (jnp.arange(EPWP - EPW, dtype=jnp.int32) % (NP - N)) + N

    def slab(ix):
        s = jnp.concatenate(
            [ix.reshape(NW, EPW), jnp.broadcast_to(pad, (NW, EPWP - EPW))],
            axis=1,
        )
        return s.reshape(NW, NCH, CH)

    srcs = slab(src)
    dsts = slab(dst)
    nf_pad = jnp.pad(n_feat, ((0, NP - N), (0, 0)))
    zrow = jnp.zeros((RPS, D), jnp.float32)
    zvec = jnp.zeros((NP,), jnp.float32)

    dego, degi = _deg(srcs, dsts, zvec)
    nout, nin, xt = _norm_call(dego, degi, nf_pad)
    p1 = _hop(xt, srcs, dsts, zrow)
    l1, xt = _comb_call(p1, nin, nout, nf_pad)
    p2 = _hop(xt, srcs, dsts, zrow)
    l2, xt = _comb_call(p2, nin, nout, nf_pad)
    p3 = _hop(xt, srcs, dsts, zrow)
    out_pad = _final_call(p3, nin, nf_pad, l1, l2, W_in, W_out)
    return out_pad[:N]


# speedup vs baseline: 11.5432x; 1.0362x over previous
import functools

import jax
import jax.numpy as jnp
from jax import lax
from jax.experimental import pallas as pl
from jax.experimental.pallas import tpu as pltpu
from jax.experimental.pallas import tpu_sc as plsc

N = 10000
E = 320000
D = 128
INIT_W = 0.9
W1 = 1.0 - INIT_W

NC = 2
NS = 16
NW = NC * NS
CH = 128
EPW = E // NW
NCH = 80
EPWP = NCH * CH
NP = 10240
RPS = NP // NS

_mesh = plsc.VectorSubcoreMesh(
    core_axis_name="c", subcore_axis_name="s", num_cores=NC, num_subcores=NS
)


@functools.partial(
    pl.kernel,
    out_type=[
        jax.ShapeDtypeStruct((NC, NP), jnp.float32),
        jax.ShapeDtypeStruct((NC, NP), jnp.float32),
    ],
    mesh=_mesh,
    scratch_types=[
        pltpu.VMEM((NCH, CH), jnp.int32),
        pltpu.VMEM((NCH, CH), jnp.int32),
        pltpu.VMEM((CH,), jnp.float32),
        pltpu.VMEM((NP // NS,), jnp.float32),
        pltpu.VMEM_SHARED((NP,), jnp.float32),
        pltpu.VMEM_SHARED((NP,), jnp.float32),
    ],
)
def _deg(srcs_hbm, dsts_hbm, dego_hbm, degi_hbm,
         idx_s, idx_d, ones_v, zb, acc_o, acc_i):
    cid = lax.axis_index("c")
    sid = lax.axis_index("s")
    wid = cid * NS + sid
    rows = pl.ds(sid * RPS, RPS)
    for j in range(RPS // 16):
        zb[pl.ds(16 * j, 16)] = jnp.zeros((16,), jnp.float32)
    pltpu.sync_copy(zb, acc_o.at[rows])
    pltpu.sync_copy(zb, acc_i.at[rows])
    for j in range(CH // 16):
        ones_v[pl.ds(16 * j, 16)] = jnp.ones((16,), jnp.float32)
    pltpu.sync_copy(srcs_hbm.at[wid], idx_s)
    pltpu.sync_copy(dsts_hbm.at[wid], idx_d)
    plsc.subcore_barrier()

    def body(c, carry):
        pltpu.sync_copy(ones_v, acc_o.at[idx_s.at[c]], add=True)
        pltpu.sync_copy(ones_v, acc_i.at[idx_d.at[c]], add=True)
        return carry

    lax.fori_loop(0, NCH, body, 0)
    plsc.subcore_barrier()
    pltpu.sync_copy(acc_o.at[rows], dego_hbm.at[cid, rows])
    pltpu.sync_copy(acc_i.at[rows], degi_hbm.at[cid, rows])


HALF = NCH // 2


@functools.partial(
    pl.kernel,
    out_type=jax.ShapeDtypeStruct((NC, NP, D), jnp.float32),
    mesh=_mesh,
    scratch_types=[
        pltpu.VMEM((HALF, CH), jnp.int32),
        pltpu.VMEM((HALF, CH), jnp.int32),
        pltpu.VMEM((2, CH, D), jnp.float32),
        pltpu.VMEM_SHARED((NP, D), jnp.float32),
        pltpu.SemaphoreType.DMA,
        pltpu.SemaphoreType.DMA,
    ],
)
def _hop(x_hbm, srcs_hbm, dsts_hbm, out_hbm,
         idx_s, idx_d, buf, acc, sem0, sem1):
    cid = lax.axis_index("c")
    sid = lax.axis_index("s")
    wid = cid * NS + sid
    rows = pl.ds(sid * RPS, RPS)

    def zrow_body(r, carry):
        for j in range(D // 16):
            buf[0, r, pl.ds(16 * j, 16)] = jnp.zeros((16,), jnp.float32)
        return carry

    lax.fori_loop(0, CH, zrow_body, 0)
    for z in range(RPS // CH):
        pltpu.sync_copy(buf.at[0], acc.at[pl.ds(sid * RPS + z * CH, CH)])
    plsc.subcore_barrier()

    for h in range(2):
        pltpu.sync_copy(srcs_hbm.at[wid, pl.ds(h * HALF, HALF)], idx_s)
        pltpu.sync_copy(dsts_hbm.at[wid, pl.ds(h * HALF, HALF)], idx_d)
        pltpu.async_copy(x_hbm.at[idx_s.at[0]], buf.at[0], sem0)

        def body(i, carry):
            c0 = 2 * i
            pltpu.async_copy(x_hbm.at[idx_s.at[c0 + 1]], buf.at[1], sem1)
            pltpu.make_async_copy(
                x_hbm.at[idx_s.at[c0]], buf.at[0], sem0).wait()
            pltpu.sync_copy(buf.at[0], acc.at[idx_d.at[c0]], add=True)

            @pl.when(i + 1 < HALF // 2)
            def _():
                pltpu.async_copy(x_hbm.at[idx_s.at[c0 + 2]], buf.at[0], sem0)

            pltpu.make_async_copy(
                x_hbm.at[idx_s.at[c0 + 1]], buf.at[1], sem1).wait()
            pltpu.sync_copy(buf.at[1], acc.at[idx_d.at[c0 + 1]], add=True)
            return carry

        lax.fori_loop(0, HALF // 2, body, 0)

    plsc.subcore_barrier()
    pltpu.sync_copy(acc.at[rows], out_hbm.at[cid, rows])


BR = 1280
G = NP // BR


def _row_spec(shape):
    if len(shape) == 3:
        return pl.BlockSpec((shape[0], BR, shape[2]), lambda i: (0, i, 0))
    if shape == (NP, 1):
        return pl.BlockSpec((BR, 1), lambda i: (i, 0))
    if shape == (NC, NP):
        return pl.BlockSpec((NC, BR), lambda i: (0, i))
    return pl.BlockSpec((BR, shape[1]), lambda i: (i, 0))


def _full_spec(shape):
    return pl.BlockSpec(shape, lambda i: tuple(0 for _ in shape))


def _norm_body(dego_ref, degi_ref, nf_ref, nout_ref, nin_ref, x0_ref):
    do = dego_ref[0, :] + dego_ref[1, :]
    di = degi_ref[0, :] + degi_ref[1, :]
    no = lax.rsqrt(jnp.maximum(do, 1.0))[:, None]
    ni = lax.rsqrt(jnp.maximum(di, 1.0))[:, None]
    nout_ref[...] = no
    nin_ref[...] = ni
    x0_ref[...] = nf_ref[...] * no


def _norm_call(dego, degi, nf):
    return pl.pallas_call(
        _norm_body,
        grid=(G,),
        in_specs=[_row_spec((NC, NP)), _row_spec((NC, NP)),
                  _row_spec((NP, D))],
        out_specs=[_row_spec((NP, 1)), _row_spec((NP, 1)),
                   _row_spec((NP, D))],
        out_shape=[
            jax.ShapeDtypeStruct((NP, 1), jnp.float32),
            jax.ShapeDtypeStruct((NP, 1), jnp.float32),
            jax.ShapeDtypeStruct((NP, D), jnp.float32),
        ],
    )(dego, degi, nf)


def _comb_body(p_ref, nin_ref, nout_ref, nf_ref, l_ref, xt_ref):
    agg = p_ref[0] + p_ref[1]
    layer = W1 * (agg * nin_ref[...]) + INIT_W * nf_ref[...]
    l_ref[...] = layer
    xt_ref[...] = layer * nout_ref[...]


def _comb_call(p, nin, nout, nf):
    return pl.pallas_call(
        _comb_body,
        grid=(G,),
        in_specs=[_row_spec((NC, NP, D)), _row_spec((NP, 1)),
                  _row_spec((NP, 1)), _row_spec((NP, D))],
        out_specs=[_row_spec((NP, D)), _row_spec((NP, D))],
        out_shape=[
            jax.ShapeDtypeStruct((NP, D), jnp.float32),
            jax.ShapeDtypeStruct((NP, D), jnp.float32),
        ],
    )(p, nin, nout, nf)


def _final_body(p_ref, nin_ref, nf_ref, l1_ref, l2_ref, win_ref, wout_ref,
                out_ref):
    f32 = jnp.float32
    bf16 = jnp.bfloat16
    agg = p_ref[0] + p_ref[1]
    l0 = nf_ref[...]
    l1 = l1_ref[...]
    l2 = l2_ref[...]
    l3 = W1 * (agg * nin_ref[...]) + INIT_W * l0

    wq_b = win_ref[0:D, :].astype(bf16)
    wk_b = win_ref[D:2 * D, :].astype(bf16)
    wv_b = win_ref[2 * D:3 * D, :].astype(bf16)
    wo_b = wout_ref[...].astype(bf16)
    scale = 1.0 / (128.0 ** 0.5)
    q0 = jnp.sum(wq_b.astype(f32), axis=1, keepdims=True)
    qsc_b = (q0 * scale).astype(bf16)
    u_row = jnp.sum(wo_b.astype(f32), axis=0, keepdims=True)

    dn_t = (((1,), (1,)), ((), ()))
    dn_v = (((1,), (0,)), ((), ()))

    def kv(l):
        l_b = l.astype(bf16)
        k_t = lax.dot_general(l_b, wk_b, dn_t, preferred_element_type=f32)
        v_t = lax.dot_general(l_b, wv_b, dn_t, preferred_element_type=f32)
        lg = lax.dot_general(k_t.astype(bf16), qsc_b, dn_v,
                             preferred_element_type=f32)
        return lg, v_t.astype(bf16).astype(f32)

    lg0, v0 = kv(l0)
    lg1, v1 = kv(l1)
    lg2, v2 = kv(l2)
    lg3, v3 = kv(l3)
    m = jnp.maximum(jnp.maximum(lg0, lg1), jnp.maximum(lg2, lg3))
    e0 = jnp.exp(lg0 - m)
    e1 = jnp.exp(lg1 - m)
    e2 = jnp.exp(lg2 - m)
    e3 = jnp.exp(lg3 - m)
    den = e0 + e1 + e2 + e3

    def wtr(e):
        return (e / den).astype(bf16).astype(f32)

    ctx = wtr(e0) * v0 + wtr(e1) * v1 + wtr(e2) * v2 + wtr(e3) * v3
    c = jnp.sum(ctx.astype(bf16).astype(f32) * u_row, axis=1, keepdims=True)
    s = c / jnp.maximum(jnp.sqrt(4.0 * (c * c)), 1e-12)
    out_ref[...] = s * (((l0 + l1) + l2) + l3)


def _final_call(p3, nin, nf, l1, l2, w_in, w_out):
    return pl.pallas_call(
        _final_body,
        grid=(G,),
        in_specs=[_row_spec((NC, NP, D)), _row_spec((NP, 1)),
                  _row_spec((NP, D)), _row_spec((NP, D)), _row_spec((NP, D)),
                  _full_spec((3 * D, D)), _full_spec((D, D))],
        out_specs=_row_spec((NP, D)),
        out_shape=jax.ShapeDtypeStruct((NP, D), jnp.float32),
    )(p3, nin, nf, l1, l2, w_in, w_out)


def kernel(n_feat, edge_index, W_in, W_out):
    src = edge_index[0]
    dst = edge_index[1]
    pad = (jnp.arange(EPWP - EPW, dtype=jnp.int32) % (NP - N)) + N

    def slab(ix):
        s = jnp.concatenate(
            [ix.reshape(NW, EPW), jnp.broadcast_to(pad, (NW, EPWP - EPW))],
            axis=1,
        )
        return s.reshape(NW, NCH, CH)

    srcs = slab(src)
    dsts = slab(dst)
    nf_pad = jnp.pad(n_feat, ((0, NP - N), (0, 0)))

    dego, degi = _deg(srcs, dsts)
    nout, nin, xt = _norm_call(dego, degi, nf_pad)
    p1 = _hop(xt, srcs, dsts)
    l1, xt = _comb_call(p1, nin, nout, nf_pad)
    p2 = _hop(xt, srcs, dsts)
    l2, xt = _comb_call(p2, nin, nout, nf_pad)
    p3 = _hop(xt, srcs, dsts)
    out_pad = _final_call(p3, nin, nf_pad, l1, l2, W_in, W_out)
    return out_pad[:N]


# speedup vs baseline: 12.2128x; 1.0580x over previous
import functools

import jax
import jax.numpy as jnp
from jax import lax
from jax.experimental import pallas as pl
from jax.experimental.pallas import tpu as pltpu
from jax.experimental.pallas import tpu_sc as plsc

N = 10000
E = 320000
D = 128
INIT_W = 0.9
W1 = 1.0 - INIT_W

NC = 2
NS = 16
NW = NC * NS
CH = 96
EPW = E // NW
NCH = 105
EPWP = NCH * CH
PC = 15
NPIECE = NCH // PC
NP = 10240
RPS = NP // NS

_mesh = plsc.VectorSubcoreMesh(
    core_axis_name="c", subcore_axis_name="s", num_cores=NC, num_subcores=NS
)


@functools.partial(
    pl.kernel,
    out_type=[
        jax.ShapeDtypeStruct((NC, NP), jnp.float32),
        jax.ShapeDtypeStruct((NC, NP), jnp.float32),
    ],
    mesh=_mesh,
    scratch_types=[
        pltpu.VMEM((NPIECE, PC, CH), jnp.int32),
        pltpu.VMEM((NPIECE, PC, CH), jnp.int32),
        pltpu.VMEM((CH,), jnp.float32),
        pltpu.VMEM((NP // NS,), jnp.float32),
        pltpu.VMEM_SHARED((NP,), jnp.float32),
        pltpu.VMEM_SHARED((NP,), jnp.float32),
    ],
)
def _deg(srcs_hbm, dsts_hbm, dego_hbm, degi_hbm,
         idx_s, idx_d, ones_v, zb, acc_o, acc_i):
    cid = lax.axis_index("c")
    sid = lax.axis_index("s")
    wid = cid * NS + sid
    rows = pl.ds(sid * RPS, RPS)
    for j in range(RPS // 16):
        zb[pl.ds(16 * j, 16)] = jnp.zeros((16,), jnp.float32)
    pltpu.sync_copy(zb, acc_o.at[rows])
    pltpu.sync_copy(zb, acc_i.at[rows])
    for j in range(CH // 16):
        ones_v[pl.ds(16 * j, 16)] = jnp.ones((16,), jnp.float32)
    pltpu.sync_copy(srcs_hbm.at[wid], idx_s)
    pltpu.sync_copy(dsts_hbm.at[wid], idx_d)
    plsc.subcore_barrier()

    def body(c, carry):
        p = c // PC
        r = lax.rem(c, PC)
        pltpu.sync_copy(ones_v, acc_o.at[idx_s.at[p, r]], add=True)
        pltpu.sync_copy(ones_v, acc_i.at[idx_d.at[p, r]], add=True)
        return carry

    lax.fori_loop(0, NCH, body, 0)
    plsc.subcore_barrier()
    pltpu.sync_copy(acc_o.at[rows], dego_hbm.at[cid, rows])
    pltpu.sync_copy(acc_i.at[rows], degi_hbm.at[cid, rows])


@functools.partial(
    pl.kernel,
    out_type=jax.ShapeDtypeStruct((NC, NP, D), jnp.float32),
    mesh=_mesh,
    scratch_types=[
        pltpu.VMEM((2, PC, CH), jnp.int32),
        pltpu.VMEM((2, PC, CH), jnp.int32),
        pltpu.VMEM((3, CH, D), jnp.float32),
        pltpu.VMEM_SHARED((NP, D), jnp.float32),
        pltpu.SemaphoreType.DMA,
        pltpu.SemaphoreType.DMA,
        pltpu.SemaphoreType.DMA,
        pltpu.SemaphoreType.DMA,
        pltpu.SemaphoreType.DMA,
        pltpu.SemaphoreType.DMA,
        pltpu.SemaphoreType.DMA,
        pltpu.SemaphoreType.DMA,
    ],
)
def _hop(x_hbm, srcs_hbm, dsts_hbm, out_hbm,
         idx_s, idx_d, buf, acc, g0, g1, g2, s0, s1, s2, i0, i1):
    gsem = (g0, g1, g2)
    ssem = (s0, s1, s2)
    isem = (i0, i1)
    cid = lax.axis_index("c")
    sid = lax.axis_index("s")
    wid = cid * NS + sid
    rows = pl.ds(sid * RPS, RPS)
    base = sid * RPS

    def zrow_body(r, carry):
        for j in range(D // 16):
            buf[0, r, pl.ds(16 * j, 16)] = jnp.zeros((16,), jnp.float32)
        return carry

    lax.fori_loop(0, CH, zrow_body, 0)
    nfull, remz = divmod(RPS, CH)
    for z in range(nfull):
        pltpu.sync_copy(buf.at[0], acc.at[pl.ds(base + z * CH, CH)])
    if remz:
        pltpu.sync_copy(buf.at[0, :remz],
                        acc.at[pl.ds(base + nfull * CH, remz)])
    plsc.subcore_barrier()

    def idx_refs(p):
        sl = p % 2
        return ((srcs_hbm.at[wid, p], idx_s.at[sl]),
                (dsts_hbm.at[wid, p], idx_d.at[sl]))

    def g_pair(c):
        p, r = divmod(c, PC)
        return x_hbm.at[idx_s.at[p % 2, r]], buf.at[c % 3]

    def s_pair(c):
        p, r = divmod(c, PC)
        return buf.at[c % 3], acc.at[idx_d.at[p % 2, r]]

    (ss, sd), (ds_, dd) = idx_refs(0)
    pltpu.sync_copy(ss, sd)
    pltpu.sync_copy(ds_, dd)
    src, dst = g_pair(0)
    pltpu.async_copy(src, dst, gsem[0])
    src, dst = g_pair(1)
    pltpu.async_copy(src, dst, gsem[1])

    for c in range(NCH):
        p, r = divmod(c, PC)
        if r == 1 and p + 1 < NPIECE:
            for hs, vs in idx_refs(p + 1):
                pltpu.async_copy(hs, vs, isem[(p + 1) % 2])
        src, dst = g_pair(c)
        pltpu.make_async_copy(src, dst, gsem[c % 3]).wait()
        src, dst = s_pair(c)
        pltpu.async_copy(src, dst, ssem[c % 3], add=True)
        if c + 2 < NCH:
            if c >= 1:
                src, dst = s_pair(c - 1)
                pltpu.make_async_copy(src, dst, ssem[(c - 1) % 3]).wait()
            if r == PC - 2:
                for hs, vs in idx_refs(p + 1):
                    pltpu.make_async_copy(hs, vs, isem[(p + 1) % 2]).wait()
            src, dst = g_pair(c + 2)
            pltpu.async_copy(src, dst, gsem[(c + 2) % 3])

    for c in range(NCH - 3, NCH):
        src, dst = s_pair(c)
        pltpu.make_async_copy(src, dst, ssem[c % 3]).wait()

    plsc.subcore_barrier()
    pltpu.sync_copy(acc.at[rows], out_hbm.at[cid, rows])


BR = 1280
G = NP // BR


def _row_spec(shape):
    if len(shape) == 3:
        return pl.BlockSpec((shape[0], BR, shape[2]), lambda i: (0, i, 0))
    if shape == (NP, 1):
        return pl.BlockSpec((BR, 1), lambda i: (i, 0))
    if shape == (NC, NP):
        return pl.BlockSpec((NC, BR), lambda i: (0, i))
    return pl.BlockSpec((BR, shape[1]), lambda i: (i, 0))


def _full_spec(shape):
    return pl.BlockSpec(shape, lambda i: tuple(0 for _ in shape))


def _norm_body(dego_ref, degi_ref, nf_ref, nout_ref, nin_ref, x0_ref):
    do = dego_ref[0, :] + dego_ref[1, :]
    di = degi_ref[0, :] + degi_ref[1, :]
    no = lax.rsqrt(jnp.maximum(do, 1.0))[:, None]
    ni = lax.rsqrt(jnp.maximum(di, 1.0))[:, None]
    nout_ref[...] = no
    nin_ref[...] = ni
    x0_ref[...] = nf_ref[...] * no


def _norm_call(dego, degi, nf):
    return pl.pallas_call(
        _norm_body,
        grid=(G,),
        in_specs=[_row_spec((NC, NP)), _row_spec((NC, NP)),
                  _row_spec((NP, D))],
        out_specs=[_row_spec((NP, 1)), _row_spec((NP, 1)),
                   _row_spec((NP, D))],
        out_shape=[
            jax.ShapeDtypeStruct((NP, 1), jnp.float32),
            jax.ShapeDtypeStruct((NP, 1), jnp.float32),
            jax.ShapeDtypeStruct((NP, D), jnp.float32),
        ],
    )(dego, degi, nf)


def _comb_body(p_ref, nin_ref, nout_ref, nf_ref, l_ref, xt_ref):
    agg = p_ref[0] + p_ref[1]
    layer = W1 * (agg * nin_ref[...]) + INIT_W * nf_ref[...]
    l_ref[...] = layer
    xt_ref[...] = layer * nout_ref[...]


def _comb_call(p, nin, nout, nf):
    return pl.pallas_call(
        _comb_body,
        grid=(G,),
        in_specs=[_row_spec((NC, NP, D)), _row_spec((NP, 1)),
                  _row_spec((NP, 1)), _row_spec((NP, D))],
        out_specs=[_row_spec((NP, D)), _row_spec((NP, D))],
        out_shape=[
            jax.ShapeDtypeStruct((NP, D), jnp.float32),
            jax.ShapeDtypeStruct((NP, D), jnp.float32),
        ],
    )(p, nin, nout, nf)


def _final_body(p_ref, nin_ref, nf_ref, l1_ref, l2_ref, win_ref, wout_ref,
                out_ref):
    f32 = jnp.float32
    bf16 = jnp.bfloat16
    agg = p_ref[0] + p_ref[1]
    l0 = nf_ref[...]
    l1 = l1_ref[...]
    l2 = l2_ref[...]
    l3 = W1 * (agg * nin_ref[...]) + INIT_W * l0

    wq_b = win_ref[0:D, :].astype(bf16)
    wk_b = win_ref[D:2 * D, :].astype(bf16)
    wv_b = win_ref[2 * D:3 * D, :].astype(bf16)
    wo_b = wout_ref[...].astype(bf16)
    scale = 1.0 / (128.0 ** 0.5)
    q0 = jnp.sum(wq_b.astype(f32), axis=1, keepdims=True)
    qsc_b = (q0 * scale).astype(bf16)
    u_row = jnp.sum(wo_b.astype(f32), axis=0, keepdims=True)

    dn_t = (((1,), (1,)), ((), ()))
    dn_v = (((1,), (0,)), ((), ()))

    def kv(l):
        l_b = l.astype(bf16)
        k_t = lax.dot_general(l_b, wk_b, dn_t, preferred_element_type=f32)
        v_t = lax.dot_general(l_b, wv_b, dn_t, preferred_element_type=f32)
        lg = lax.dot_general(k_t.astype(bf16), qsc_b, dn_v,
                             preferred_element_type=f32)
        return lg, v_t.astype(bf16).astype(f32)

    lg0, v0 = kv(l0)
    lg1, v1 = kv(l1)
    lg2, v2 = kv(l2)
    lg3, v3 = kv(l3)
    m = jnp.maximum(jnp.maximum(lg0, lg1), jnp.maximum(lg2, lg3))
    e0 = jnp.exp(lg0 - m)
    e1 = jnp.exp(lg1 - m)
    e2 = jnp.exp(lg2 - m)
    e3 = jnp.exp(lg3 - m)
    den = e0 + e1 + e2 + e3

    def wtr(e):
        return (e / den).astype(bf16).astype(f32)

    ctx = wtr(e0) * v0 + wtr(e1) * v1 + wtr(e2) * v2 + wtr(e3) * v3
    c = jnp.sum(ctx.astype(bf16).astype(f32) * u_row, axis=1, keepdims=True)
    s = c / jnp.maximum(jnp.sqrt(4.0 * (c * c)), 1e-12)
    out_ref[...] = s * (((l0 + l1) + l2) + l3)


def _final_call(p3, nin, nf, l1, l2, w_in, w_out):
    return pl.pallas_call(
        _final_body,
        grid=(G,),
        in_specs=[_row_spec((NC, NP, D)), _row_spec((NP, 1)),
                  _row_spec((NP, D)), _row_spec((NP, D)), _row_spec((NP, D)),
                  _full_spec((3 * D, D)), _full_spec((D, D))],
        out_specs=_row_spec((NP, D)),
        out_shape=jax.ShapeDtypeStruct((NP, D), jnp.float32),
    )(p3, nin, nf, l1, l2, w_in, w_out)


def kernel(n_feat, edge_index, W_in, W_out):
    src = edge_index[0]
    dst = edge_index[1]
    pad = (jnp.arange(EPWP - EPW, dtype=jnp.int32) % (NP - N)) + N

    def slab(ix):
        s = jnp.concatenate(
            [ix.reshape(NW, EPW), jnp.broadcast_to(pad, (NW, EPWP - EPW))],
            axis=1,
        )
        return s.reshape(NW, NPIECE, PC, CH)

    srcs = slab(src)
    dsts = slab(dst)
    nf_pad = jnp.pad(n_feat, ((0, NP - N), (0, 0)))

    dego, degi = _deg(srcs, dsts)
    nout, nin, xt = _norm_call(dego, degi, nf_pad)
    p1 = _hop(xt, srcs, dsts)
    l1, xt = _comb_call(p1, nin, nout, nf_pad)
    p2 = _hop(xt, srcs, dsts)
    l2, xt = _comb_call(p2, nin, nout, nf_pad)
    p3 = _hop(xt, srcs, dsts)
    out_pad = _final_call(p3, nin, nf_pad, l1, l2, W_in, W_out)
    return out_pad[:N]


# speedup vs baseline: 12.6736x; 1.0377x over previous
import functools

import jax
import jax.numpy as jnp
from jax import lax
from jax.experimental import pallas as pl
from jax.experimental.pallas import tpu as pltpu
from jax.experimental.pallas import tpu_sc as plsc

N = 10000
E = 320000
D = 128
INIT_W = 0.9
W1 = 1.0 - INIT_W

NC = 2
NS = 16
NW = NC * NS
CH = 96
EPW = E // NW
NCH = 105
EPWP = NCH * CH
PC = 15
NPIECE = NCH // PC
NP = 10240
RPS = NP // NS

_mesh = plsc.VectorSubcoreMesh(
    core_axis_name="c", subcore_axis_name="s", num_cores=NC, num_subcores=NS
)


@functools.partial(
    pl.kernel,
    out_type=[
        jax.ShapeDtypeStruct((NC, NP), jnp.float32),
        jax.ShapeDtypeStruct((NC, NP), jnp.float32),
    ],
    mesh=_mesh,
    scratch_types=[
        pltpu.VMEM((NPIECE, PC, CH), jnp.int32),
        pltpu.VMEM((NPIECE, PC, CH), jnp.int32),
        pltpu.VMEM((CH,), jnp.float32),
        pltpu.VMEM((NP // NS,), jnp.float32),
        pltpu.VMEM_SHARED((NP,), jnp.float32),
        pltpu.VMEM_SHARED((NP,), jnp.float32),
        pltpu.SemaphoreType.DMA,
        pltpu.SemaphoreType.DMA,
    ],
)
def _deg(srcs_hbm, dsts_hbm, dego_hbm, degi_hbm,
         idx_s, idx_d, ones_v, zb, acc_o, acc_i, so, si):
    cid = lax.axis_index("c")
    sid = lax.axis_index("s")
    wid = cid * NS + sid
    rows = pl.ds(sid * RPS, RPS)
    for j in range(RPS // 16):
        zb[pl.ds(16 * j, 16)] = jnp.zeros((16,), jnp.float32)
    pltpu.sync_copy(zb, acc_o.at[rows])
    pltpu.sync_copy(zb, acc_i.at[rows])
    for j in range(CH // 16):
        ones_v[pl.ds(16 * j, 16)] = jnp.ones((16,), jnp.float32)
    pltpu.sync_copy(srcs_hbm.at[wid], idx_s)
    pltpu.sync_copy(dsts_hbm.at[wid], idx_d)
    plsc.subcore_barrier()

    def fire(c, carry):
        p = c // PC
        r = lax.rem(c, PC)
        pltpu.async_copy(ones_v, acc_o.at[idx_s.at[p, r]], so, add=True)
        pltpu.async_copy(ones_v, acc_i.at[idx_d.at[p, r]], si, add=True)
        return carry

    lax.fori_loop(0, NCH, fire, 0)

    def drain(c, carry):
        p = c // PC
        r = lax.rem(c, PC)
        pltpu.make_async_copy(ones_v, acc_o.at[idx_s.at[p, r]], so).wait()
        pltpu.make_async_copy(ones_v, acc_i.at[idx_d.at[p, r]], si).wait()
        return carry

    lax.fori_loop(0, NCH, drain, 0)
    plsc.subcore_barrier()
    pltpu.sync_copy(acc_o.at[rows], dego_hbm.at[cid, rows])
    pltpu.sync_copy(acc_i.at[rows], degi_hbm.at[cid, rows])


@functools.partial(
    pl.kernel,
    out_type=jax.ShapeDtypeStruct((NC, NP, D), jnp.float32),
    mesh=_mesh,
    scratch_types=[
        pltpu.VMEM((2, PC, CH), jnp.int32),
        pltpu.VMEM((2, PC, CH), jnp.int32),
        pltpu.VMEM((3, CH, D), jnp.float32),
        pltpu.VMEM_SHARED((NP, D), jnp.float32),
        pltpu.SemaphoreType.DMA,
        pltpu.SemaphoreType.DMA,
        pltpu.SemaphoreType.DMA,
        pltpu.SemaphoreType.DMA,
        pltpu.SemaphoreType.DMA,
        pltpu.SemaphoreType.DMA,
        pltpu.SemaphoreType.DMA,
        pltpu.SemaphoreType.DMA,
    ],
)
def _hop(x_hbm, srcs_hbm, dsts_hbm, out_hbm,
         idx_s, idx_d, buf, acc, g0, g1, g2, s0, s1, s2, i0, i1):
    gsem = (g0, g1, g2)
    ssem = (s0, s1, s2)
    isem = (i0, i1)
    cid = lax.axis_index("c")
    sid = lax.axis_index("s")
    wid = cid * NS + sid
    rows = pl.ds(sid * RPS, RPS)
    base = sid * RPS

    def zrow_body(r, carry):
        for j in range(D // 16):
            buf[0, r, pl.ds(16 * j, 16)] = jnp.zeros((16,), jnp.float32)
        return carry

    lax.fori_loop(0, CH, zrow_body, 0)
    nfull, remz = divmod(RPS, CH)
    for z in range(nfull):
        pltpu.sync_copy(buf.at[0], acc.at[pl.ds(base + z * CH, CH)])
    if remz:
        pltpu.sync_copy(buf.at[0, :remz],
                        acc.at[pl.ds(base + nfull * CH, remz)])
    plsc.subcore_barrier()

    def idx_refs(p):
        sl = p % 2
        return ((srcs_hbm.at[wid, p], idx_s.at[sl]),
                (dsts_hbm.at[wid, p], idx_d.at[sl]))

    def g_pair(c):
        p, r = divmod(c, PC)
        return x_hbm.at[idx_s.at[p % 2, r]], buf.at[c % 3]

    def s_pair(c):
        p, r = divmod(c, PC)
        return buf.at[c % 3], acc.at[idx_d.at[p % 2, r]]

    (ss, sd), (ds_, dd) = idx_refs(0)
    pltpu.sync_copy(ss, sd)
    pltpu.sync_copy(ds_, dd)
    src, dst = g_pair(0)
    pltpu.async_copy(src, dst, gsem[0])
    src, dst = g_pair(1)
    pltpu.async_copy(src, dst, gsem[1])

    for c in range(NCH):
        p, r = divmod(c, PC)
        if r == 1 and p + 1 < NPIECE:
            for hs, vs in idx_refs(p + 1):
                pltpu.async_copy(hs, vs, isem[(p + 1) % 2])
        src, dst = g_pair(c)
        pltpu.make_async_copy(src, dst, gsem[c % 3]).wait()
        src, dst = s_pair(c)
        pltpu.async_copy(src, dst, ssem[c % 3], add=True)
        if c + 2 < NCH:
            if c >= 1:
                src, dst = s_pair(c - 1)
                pltpu.make_async_copy(src, dst, ssem[(c - 1) % 3]).wait()
            if r == PC - 2:
                for hs, vs in idx_refs(p + 1):
                    pltpu.make_async_copy(hs, vs, isem[(p + 1) % 2]).wait()
            src, dst = g_pair(c + 2)
            pltpu.async_copy(src, dst, gsem[(c + 2) % 3])

    for c in range(NCH - 3, NCH):
        src, dst = s_pair(c)
        pltpu.make_async_copy(src, dst, ssem[c % 3]).wait()

    plsc.subcore_barrier()
    pltpu.sync_copy(acc.at[rows], out_hbm.at[cid, rows])


BR = 1280
G = NP // BR


def _row_spec(shape):
    if len(shape) == 3:
        return pl.BlockSpec((shape[0], BR, shape[2]), lambda i: (0, i, 0))
    if shape == (NP, 1):
        return pl.BlockSpec((BR, 1), lambda i: (i, 0))
    if shape == (NC, NP):
        return pl.BlockSpec((NC, BR), lambda i: (0, i))
    return pl.BlockSpec((BR, shape[1]), lambda i: (i, 0))


def _full_spec(shape):
    return pl.BlockSpec(shape, lambda i: tuple(0 for _ in shape))


def _norm_body(dego_ref, degi_ref, nf_ref, nout_ref, nin_ref, x0_ref):
    do = dego_ref[0, :] + dego_ref[1, :]
    di = degi_ref[0, :] + degi_ref[1, :]
    no = lax.rsqrt(jnp.maximum(do, 1.0))[:, None]
    ni = lax.rsqrt(jnp.maximum(di, 1.0))[:, None]
    nout_ref[...] = no
    nin_ref[...] = ni
    x0_ref[...] = nf_ref[...] * no


def _norm_call(dego, degi, nf):
    return pl.pallas_call(
        _norm_body,
        grid=(G,),
        in_specs=[_row_spec((NC, NP)), _row_spec((NC, NP)),
                  _row_spec((NP, D))],
        out_specs=[_row_spec((NP, 1)), _row_spec((NP, 1)),
                   _row_spec((NP, D))],
        out_shape=[
            jax.ShapeDtypeStruct((NP, 1), jnp.float32),
            jax.ShapeDtypeStruct((NP, 1), jnp.float32),
            jax.ShapeDtypeStruct((NP, D), jnp.float32),
        ],
    )(dego, degi, nf)


def _comb_body(p_ref, nin_ref, nout_ref, nf_ref, l_ref, xt_ref):
    agg = p_ref[0] + p_ref[1]
    layer = W1 * (agg * nin_ref[...]) + INIT_W * nf_ref[...]
    l_ref[...] = layer
    xt_ref[...] = layer * nout_ref[...]


def _comb_call(p, nin, nout, nf):
    return pl.pallas_call(
        _comb_body,
        grid=(G,),
        in_specs=[_row_spec((NC, NP, D)), _row_spec((NP, 1)),
                  _row_spec((NP, 1)), _row_spec((NP, D))],
        out_specs=[_row_spec((NP, D)), _row_spec((NP, D))],
        out_shape=[
            jax.ShapeDtypeStruct((NP, D), jnp.float32),
            jax.ShapeDtypeStruct((NP, D), jnp.float32),
        ],
    )(p, nin, nout, nf)


def _final_body(p_ref, nin_ref, nf_ref, l1_ref, l2_ref, win_ref, wout_ref,
                out_ref):
    f32 = jnp.float32
    bf16 = jnp.bfloat16
    agg = p_ref[0] + p_ref[1]
    l0 = nf_ref[...]
    l1 = l1_ref[...]
    l2 = l2_ref[...]
    l3 = W1 * (agg * nin_ref[...]) + INIT_W * l0

    wq_b = win_ref[0:D, :].astype(bf16)
    wk_b = win_ref[D:2 * D, :].astype(bf16)
    wv_b = win_ref[2 * D:3 * D, :].astype(bf16)
    wo_b = wout_ref[...].astype(bf16)
    scale = 1.0 / (128.0 ** 0.5)
    q0 = jnp.sum(wq_b.astype(f32), axis=1, keepdims=True)
    qsc_b = (q0 * scale).astype(bf16)
    u_row = jnp.sum(wo_b.astype(f32), axis=0, keepdims=True)

    dn_t = (((1,), (1,)), ((), ()))
    dn_v = (((1,), (0,)), ((), ()))

    def kv(l):
        l_b = l.astype(bf16)
        k_t = lax.dot_general(l_b, wk_b, dn_t, preferred_element_type=f32)
        v_t = lax.dot_general(l_b, wv_b, dn_t, preferred_element_type=f32)
        lg = lax.dot_general(k_t.astype(bf16), qsc_b, dn_v,
                             preferred_element_type=f32)
        return lg, v_t.astype(bf16).astype(f32)

    lg0, v0 = kv(l0)
    lg1, v1 = kv(l1)
    lg2, v2 = kv(l2)
    lg3, v3 = kv(l3)
    m = jnp.maximum(jnp.maximum(lg0, lg1), jnp.maximum(lg2, lg3))
    e0 = jnp.exp(lg0 - m)
    e1 = jnp.exp(lg1 - m)
    e2 = jnp.exp(lg2 - m)
    e3 = jnp.exp(lg3 - m)
    den = e0 + e1 + e2 + e3

    def wtr(e):
        return (e / den).astype(bf16).astype(f32)

    ctx = wtr(e0) * v0 + wtr(e1) * v1 + wtr(e2) * v2 + wtr(e3) * v3
    c = jnp.sum(ctx.astype(bf16).astype(f32) * u_row, axis=1, keepdims=True)
    s = c / jnp.maximum(jnp.sqrt(4.0 * (c * c)), 1e-12)
    out_ref[...] = s * (((l0 + l1) + l2) + l3)


def _final_call(p3, nin, nf, l1, l2, w_in, w_out):
    return pl.pallas_call(
        _final_body,
        grid=(G,),
        in_specs=[_row_spec((NC, NP, D)), _row_spec((NP, 1)),
                  _row_spec((NP, D)), _row_spec((NP, D)), _row_spec((NP, D)),
                  _full_spec((3 * D, D)), _full_spec((D, D))],
        out_specs=_row_spec((NP, D)),
        out_shape=jax.ShapeDtypeStruct((NP, D), jnp.float32),
    )(p3, nin, nf, l1, l2, w_in, w_out)


def kernel(n_feat, edge_index, W_in, W_out):
    src = edge_index[0]
    dst = edge_index[1]
    pad = (jnp.arange(EPWP - EPW, dtype=jnp.int32) % (NP - N)) + N

    def slab(ix):
        s = jnp.concatenate(
            [ix.reshape(NW, EPW), jnp.broadcast_to(pad, (NW, EPWP - EPW))],
            axis=1,
        )
        return s.reshape(NW, NPIECE, PC, CH)

    srcs = slab(src)
    dsts = slab(dst)
    nf_pad = jnp.pad(n_feat, ((0, NP - N), (0, 0)))

    dego, degi = _deg(srcs, dsts)
    nout, nin, xt = _norm_call(dego, degi, nf_pad)
    p1 = _hop(xt, srcs, dsts)
    l1, xt = _comb_call(p1, nin, nout, nf_pad)
    p2 = _hop(xt, srcs, dsts)
    l2, xt = _comb_call(p2, nin, nout, nf_pad)
    p3 = _hop(xt, srcs, dsts)
    out_pad = _final_call(p3, nin, nf_pad, l1, l2, W_in, W_out)
    return out_pad[:N]
